# baseline (speedup 1.0000x reference)
import functools

import jax
import jax.numpy as jnp
from jax import lax
from jax.experimental import pallas as pl
from jax.experimental.pallas import tpu as pltpu
from jax.experimental.pallas import tpu_sc as plsc

BATCH = 16384
SEQ = 50
EMBED_DIM = 64
VOCAB_ROWS = 1000000

_info = plsc.get_sparse_core_info()
_NC = _info.num_cores
_NS = _info.num_subcores
_NW = _NC * _NS
_RPW = BATCH // _NW
_BTPW = _RPW // 128

_NBUF = 4
_LEAD = 2
_NCHUNK = SEQ * _BTPW
_NGROUP = _NCHUNK // _NBUF

_mesh = plsc.VectorSubcoreMesh(core_axis_name="c", subcore_axis_name="s")


@functools.partial(
    pl.kernel,
    mesh=_mesh,
    out_type=jax.ShapeDtypeStruct((SEQ, 8, BATCH // 128, 8, 128),
                                  jnp.float32),
    scratch_types=[
        pltpu.VMEM((_RPW, SEQ), jnp.int32),
        pltpu.VMEM((SEQ, _RPW), jnp.int32),
        pltpu.VMEM((_NBUF, 128, EMBED_DIM), jnp.float32),
        pltpu.VMEM((_NBUF, 8, 8, 128), jnp.float32),
        pltpu.SemaphoreType.DMA,
        pltpu.SemaphoreType.DMA,
        pltpu.SemaphoreType.DMA,
        pltpu.SemaphoreType.DMA,
        pltpu.SemaphoreType.DMA,
        pltpu.SemaphoreType.DMA,
        pltpu.SemaphoreType.DMA,
        pltpu.SemaphoreType.DMA,
    ],
    compiler_params=pltpu.CompilerParams(use_tc_tiling_on_sc=False,
                                        needs_layout_passes=False),
)
def _emb_lookup(idx_hbm, table_hbm, out_hbm, idx_raw, idx_t, rows, tiles,
                g0, g1, g2, g3, w0, w1, w2, w3):
    gsems = (g0, g1, g2, g3)
    wsems = (w0, w1, w2, w3)
    wid = lax.axis_index("s") * _NC + lax.axis_index("c")
    row0 = wid * _RPW
    bt0 = wid * _BTPW

    pltpu.sync_copy(idx_hbm.at[pl.ds(row0, _RPW), :], idx_raw)

    iota = lax.iota(jnp.int32, 16)

    def tr_idx(v, carry):
        rvec = v * 16 + iota
        for s in range(SEQ):
            val = plsc.load_gather(idx_raw, [rvec, jnp.full((16,), s, jnp.int32)])
            idx_t[s, pl.ds(v * 16, 16)] = val + val
        return carry

    lax.fori_loop(0, _RPW // 16, tr_idx, 0)

    def start_gather(slot, chunk):
        s = chunk % SEQ
        btl = chunk // SEQ
        pltpu.async_copy(
            table_hbm.at[idx_t.at[s, pl.ds(btl * 128, 128)]],
            rows.at[slot], gsems[slot])

    def wait_gather(slot):
        pltpu.make_async_copy(
            table_hbm.at[idx_t.at[0, pl.ds(0, 128)]],
            rows.at[slot], gsems[slot]).wait()

    def start_wb(slot, chunk):
        s = chunk % SEQ
        btl = chunk // SEQ
        pltpu.async_copy(
            tiles.at[slot],
            out_hbm.at[s, :, bt0 + btl, :, :], wsems[slot])

    def wait_wb(slot):
        pltpu.make_async_copy(
            tiles.at[slot],
            out_hbm.at[0, :, 0, :, :], wsems[slot]).wait()

    rowsel = [v * 16 + iota for v in range(8)]

    def transpose_chunk(slot):
        src = rows.at[slot]
        for dt in range(8):
            for ds in range(8):
                col = jnp.full((16,), dt * 8 + ds, jnp.int32)
                for v in range(8):
                    val = plsc.load_gather(src, [rowsel[v], col])
                    tiles[slot, dt, ds, pl.ds(v * 16, 16)] = val

    for b in range(_LEAD):
        start_gather(b, b)

    def group(g, carry):
        for b in range(_NBUF):
            c = g * _NBUF + b
            wait_gather(b)

            @pl.when(c >= _NBUF)
            def _():
                wait_wb(b)

            transpose_chunk(b)
            start_wb(b, c)
            j = c + _LEAD

            @pl.when(j < _NCHUNK)
            def _():
                start_gather((b + _LEAD) % _NBUF, j)
        return carry

    lax.fori_loop(0, _NGROUP, group, 0)

    for b in range(_NBUF):
        wait_wb(b)


def kernel(questions_tensor, table):
    tpad = jnp.pad(table, ((0, 0), (0, 64)))
    t2 = tpad.reshape(2 * VOCAB_ROWS, EMBED_DIM)
    out5 = _emb_lookup(questions_tensor, t2)
    return out5.transpose(2, 4, 0, 1, 3).reshape(BATCH, SEQ, EMBED_DIM)


# speedup vs baseline: 1.6288x; 1.6288x over previous
import functools

import jax
import jax.numpy as jnp
from jax import lax
from jax.experimental import pallas as pl
from jax.experimental.pallas import tpu as pltpu
from jax.experimental.pallas import tpu_sc as plsc

BATCH = 16384
SEQ = 50
EMBED_DIM = 64
VOCAB_ROWS = 1000000

_info = plsc.get_sparse_core_info()
_NC = _info.num_cores
_NS = _info.num_subcores
_NW = _NC * _NS
_RPW = BATCH // _NW

_NBUF = 4
_LEAD = 2
_CROWS = 8
_NCHUNK = _RPW // _CROWS
_NGROUP = _NCHUNK // _NBUF

_mesh = plsc.VectorSubcoreMesh(core_axis_name="c", subcore_axis_name="s")


@functools.partial(
    pl.kernel,
    mesh=_mesh,
    out_type=jax.ShapeDtypeStruct((BATCH, SEQ, EMBED_DIM), jnp.float32),
    scratch_types=[
        pltpu.VMEM((_RPW, SEQ), jnp.int32),
        pltpu.VMEM((_NBUF, _CROWS, SEQ, EMBED_DIM), jnp.float32),
        pltpu.SemaphoreType.DMA,
        pltpu.SemaphoreType.DMA,
        pltpu.SemaphoreType.DMA,
        pltpu.SemaphoreType.DMA,
        pltpu.SemaphoreType.DMA,
        pltpu.SemaphoreType.DMA,
        pltpu.SemaphoreType.DMA,
        pltpu.SemaphoreType.DMA,
    ],
    compiler_params=pltpu.CompilerParams(use_tc_tiling_on_sc=False),
)
def _emb_lookup(idx_hbm, table_hbm, out_hbm, idx_all, rows,
                g0, g1, g2, g3, w0, w1, w2, w3):
    gsems = (g0, g1, g2, g3)
    wsems = (w0, w1, w2, w3)
    wid = lax.axis_index("s") * _NC + lax.axis_index("c")
    row0 = wid * _RPW

    pltpu.sync_copy(idx_hbm.at[pl.ds(row0, _RPW), :], idx_all)

    def start_gather(slot, chunk):
        for r in range(_CROWS):
            pltpu.async_copy(
                table_hbm.at[idx_all.at[chunk * _CROWS + r, :]],
                rows.at[slot].at[r], gsems[slot])

    def wait_gather(slot):
        for r in range(_CROWS):
            pltpu.make_async_copy(
                table_hbm.at[idx_all.at[0, :]],
                rows.at[slot].at[r], gsems[slot]).wait()

    def start_wb(slot, chunk):
        pltpu.async_copy(
            rows.at[slot],
            out_hbm.at[pl.ds(row0 + chunk * _CROWS, _CROWS), :, :],
            wsems[slot])

    def wait_wb(slot):
        pltpu.make_async_copy(
            rows.at[slot],
            out_hbm.at[pl.ds(row0, _CROWS), :, :], wsems[slot]).wait()

    for b in range(_LEAD):
        start_gather(b, b)

    def group(g, carry):
        for b in range(_NBUF):
            i = g * _NBUF + b
            wait_gather(b)
            start_wb(b, i)
            j = i + _LEAD
            sj = (b + _LEAD) % _NBUF

            @pl.when(j < _NCHUNK)
            def _():
                @pl.when(j >= _NBUF)
                def _():
                    wait_wb(sj)
                start_gather(sj, j)
        return carry

    lax.fori_loop(0, _NGROUP, group, 0)

    for b in range(_NBUF):
        wait_wb(b)


def kernel(questions_tensor, table):
    q2 = questions_tensor * 2
    tpad = jnp.pad(table, ((0, 0), (0, 64)))
    t2 = tpad.reshape(2 * VOCAB_ROWS, EMBED_DIM)
    return _emb_lookup(q2, t2)
